# baseline (speedup 1.0000x reference)
import dataclasses
import functools

import jax
import jax.numpy as jnp
from jax import lax
from jax.experimental import pallas as pl
from jax.experimental.pallas import tpu as pltpu
from jax.experimental.pallas import tpu_sc as plsc

H = 100
W = 100
D = 256
B = 8
PLANE_ROWS = 2 * D
OUT_ROWS = B * PLANE_ROWS
ROW_LEN = H * W
C_PER_SUB = 16
PIECE = 8


def _sc_body(rowt_ref, colt_ref, idxdiv_ref, idxmod_ref, out_ref,
             tbl_v, idx_v, buf, sem):
    core = lax.axis_index("c")
    sub = lax.axis_index("s")

    @pl.when(core == 0)
    def _():
        pltpu.sync_copy(rowt_ref.at[pl.ds(sub * C_PER_SUB * H, C_PER_SUB * H)],
                        tbl_v)
        pltpu.sync_copy(idxdiv_ref, idx_v)

    @pl.when(core == 1)
    def _():
        pltpu.sync_copy(colt_ref.at[pl.ds(sub * C_PER_SUB * W, C_PER_SUB * W)],
                        tbl_v)
        pltpu.sync_copy(idxmod_ref, idx_v)

    for p in range(C_PER_SUB // PIECE):
        @pl.loop(0, PIECE)
        def _(k):
            t = p * PIECE + k

            @pl.loop(0, ROW_LEN // 16)
            def _(ch):
                idx = idx_v[pl.ds(ch * 16, 16)] + t * H
                v = plsc.load_gather(tbl_v, [idx])
                buf[pl.ds(k * ROW_LEN + ch * 16, 16)] = v

        copies = []
        for b in range(B):
            c0 = core * D + sub * C_PER_SUB + p * PIECE
            base = (b * PLANE_ROWS + c0) * ROW_LEN
            copies.append(pltpu.async_copy(
                buf, out_ref.at[pl.ds(base, PIECE * ROW_LEN)], sem))
        for cp in copies:
            cp.wait()


def kernel(x, row_embed, col_embed):
    f32 = jnp.float32
    row_t = row_embed.T.reshape(-1)
    col_t = col_embed.T.reshape(-1)
    lanes = jnp.arange(ROW_LEN, dtype=jnp.int32)
    idx_div = lanes // W
    idx_mod = lanes % W
    mesh = plsc.VectorSubcoreMesh(core_axis_name="c", subcore_axis_name="s")
    cp = pltpu.CompilerParams()
    if "needs_layout_passes" in pltpu.CompilerParams.__dataclass_fields__:
        cp = dataclasses.replace(cp, needs_layout_passes=False)
    run = pl.kernel(
        _sc_body,
        out_type=jax.ShapeDtypeStruct((OUT_ROWS * ROW_LEN,), f32),
        mesh=mesh,
        scratch_types=[
            pltpu.VMEM((C_PER_SUB * H,), f32),
            pltpu.VMEM((ROW_LEN,), jnp.int32),
            pltpu.VMEM((PIECE * ROW_LEN,), f32),
            pltpu.SemaphoreType.DMA,
        ],
        compiler_params=cp,
    )
    out = run(row_t, col_t, idx_div, idx_mod)
    return out.reshape(B, PLANE_ROWS, H, W)
